# Initial kernel scaffold; baseline (speedup 1.0000x reference)
import jax
import jax.numpy as jnp
from jax.experimental import pallas as pl


def kernel(x, edge_index, W1, a_s1, a_d1, b1, W2, a_s2, a_d2, b2, W3, a_s3, a_d3, b3, Wih_f, Whh_f, bih_f, bhh_f, Wih_b, Whh_b, bih_b, bhh_b, Wmu, bmu, Wlv, blv, Wpi, bpi):
    raise NotImplementedError("write your pallas kernel here")


# speedup vs baseline: 187.5411x; 187.5411x over previous
import functools

import jax
import jax.numpy as jnp
from jax.experimental import pallas as pl

_N = 512
_G = 32
_E = 8192
_ECHUNK = 2048

_INTERP = False


def _adj_kernel(src_ref, dst_ref, m_ref):
    c = pl.program_id(0)
    lane = jax.lax.broadcasted_iota(jnp.int32, (_ECHUNK, _N), 1)
    s_hot = (src_ref[...] == lane).astype(jnp.bfloat16)
    d_hot = (dst_ref[...] == lane).astype(jnp.bfloat16)
    acc = jax.lax.dot_general(
        d_hot, s_hot, (((0,), (0,)), ((), ())),
        preferred_element_type=jnp.float32)

    @pl.when(c == 0)
    def _():
        r = jax.lax.broadcasted_iota(jnp.int32, (_N, _N), 0)
        l = jax.lax.broadcasted_iota(jnp.int32, (_N, _N), 1)
        m_ref[...] = (r == l).astype(jnp.float32)

    m_ref[...] += acc


def _build_adj(edge_index):
    src = edge_index[0].reshape(_E, 1)
    dst = edge_index[1].reshape(_E, 1)
    grid = _E // _ECHUNK
    return pl.pallas_call(
        _adj_kernel,
        grid=(grid,),
        in_specs=[
            pl.BlockSpec((_ECHUNK, 1), lambda c: (c, 0)),
            pl.BlockSpec((_ECHUNK, 1), lambda c: (c, 0)),
        ],
        out_specs=pl.BlockSpec((_N, _N), lambda c: (0, 0)),
        out_shape=jax.ShapeDtypeStruct((_N, _N), jnp.float32),
        interpret=_INTERP,
    )(src, dst)


def _gat_kernel(x_ref, m_ref, w_ref, as_ref, ad_ref, b_ref, o_ref,
                *, heads, out_ch, pool):
    xg = x_ref[0]
    h = jnp.dot(xg, w_ref[...], preferred_element_type=jnp.float32)
    asrc = jnp.dot(h, as_ref[...], preferred_element_type=jnp.float32,
                   precision=jax.lax.Precision.HIGHEST)
    adst = jnp.dot(h, ad_ref[...], preferred_element_type=jnp.float32,
                   precision=jax.lax.Precision.HIGHEST)
    asrc_t = jnp.transpose(asrc)
    mm = m_ref[...]
    outs = []
    for k in range(heads):
        logit = adst[:, k:k + 1] + asrc_t[k:k + 1, :]
        alpha = jnp.where(logit >= 0.0, jnp.minimum(logit, 60.0),
                          0.2 * logit)
        e = jnp.exp(alpha) * mm
        rdenom = 1.0 / (jnp.sum(e, axis=1, keepdims=True) + 1e-16)
        hk = h[:, k * out_ch:(k + 1) * out_ch]
        outs.append(
            jnp.dot(e, hk, preferred_element_type=jnp.float32,
                    precision=jax.lax.Precision.HIGHEST) * rdenom)
    out = outs[0] if heads == 1 else jnp.concatenate(outs, axis=1)
    out = jnp.maximum(out + b_ref[...], 0.0)
    if pool:
        o_ref[0] = jnp.sum(out, axis=0, keepdims=True)
    else:
        o_ref[0] = out


def _gat_layer(x, m, w, a_src, a_dst, b, heads, out_ch, pool):
    cin = x.shape[-1]
    cout = heads * out_ch
    eye = jnp.eye(heads, dtype=jnp.float32)
    a_s = (a_src[:, :, None] * eye[:, None, :]).reshape(cout, heads)
    a_d = (a_dst[:, :, None] * eye[:, None, :]).reshape(cout, heads)
    odim = out_ch if heads == 1 else cout
    oshape = (_G, 1, odim) if pool else (_G, _N, odim)
    ospec = (pl.BlockSpec((1, 1, odim), lambda g: (g, 0, 0)) if pool
             else pl.BlockSpec((1, _N, odim), lambda g: (g, 0, 0)))
    out = pl.pallas_call(
        functools.partial(_gat_kernel, heads=heads, out_ch=out_ch, pool=pool),
        grid=(_G,),
        in_specs=[
            pl.BlockSpec((1, _N, cin), lambda g: (g, 0, 0)),
            pl.BlockSpec((_N, _N), lambda g: (0, 0)),
            pl.BlockSpec((cin, cout), lambda g: (0, 0)),
            pl.BlockSpec((cout, heads), lambda g: (0, 0)),
            pl.BlockSpec((cout, heads), lambda g: (0, 0)),
            pl.BlockSpec((1, cout), lambda g: (0, 0)),
        ],
        out_specs=ospec,
        out_shape=jax.ShapeDtypeStruct(oshape, jnp.float32),
        interpret=_INTERP,
    )(x, m, w, a_s, a_d, b.reshape(1, cout))
    return out.reshape(_G, odim) if pool else out


def _lstm_kernel(seq_ref, wih_f_ref, whh_f_ref, bf_ref,
                 wih_b_ref, whh_b_ref, bb_ref, wout_ref, bout_ref, o_ref):
    bs = 4
    hdim = 256
    h_f = jnp.zeros((bs, hdim), jnp.float32)
    c_f = jnp.zeros((bs, hdim), jnp.float32)
    h_b = jnp.zeros((bs, hdim), jnp.float32)
    c_b = jnp.zeros((bs, hdim), jnp.float32)

    def cell(xt, h, c, wih, whh, bias):
        g = (jnp.dot(xt, wih, preferred_element_type=jnp.float32)
             + jnp.dot(h, whh, preferred_element_type=jnp.float32) + bias)
        i = jax.nn.sigmoid(g[:, 0:hdim])
        f = jax.nn.sigmoid(g[:, hdim:2 * hdim])
        gg = jnp.tanh(g[:, 2 * hdim:3 * hdim])
        o = jax.nn.sigmoid(g[:, 3 * hdim:4 * hdim])
        c = f * c + i * gg
        h = o * jnp.tanh(c)
        return h, c

    for t in range(8):
        h_f, c_f = cell(seq_ref[t], h_f, c_f,
                        wih_f_ref[...], whh_f_ref[...], bf_ref[...])
        h_b, c_b = cell(seq_ref[7 - t], h_b, c_b,
                        wih_b_ref[...], whh_b_ref[...], bb_ref[...])

    temporal = jnp.concatenate([h_f, h_b], axis=1)
    o_ref[...] = (jnp.dot(temporal, wout_ref[...],
                          preferred_element_type=jnp.float32)
                  + bout_ref[...])


def _lstm_heads(gemb, wih_f, whh_f, b_f, wih_b, whh_b, b_b, wout, bout):
    seq = jnp.transpose(gemb.reshape(4, 8, 64), (1, 0, 2))
    odim = wout.shape[1]
    return pl.pallas_call(
        _lstm_kernel,
        out_shape=jax.ShapeDtypeStruct((4, odim), jnp.float32),
        interpret=_INTERP,
    )(seq, wih_f, whh_f, b_f.reshape(1, -1),
      wih_b, whh_b, b_b.reshape(1, -1), wout, bout.reshape(1, -1))


def kernel(x, edge_index, W1, a_s1, a_d1, b1, W2, a_s2, a_d2, b2,
           W3, a_s3, a_d3, b3, Wih_f, Whh_f, bih_f, bhh_f,
           Wih_b, Whh_b, bih_b, bhh_b, Wmu, bmu, Wlv, blv, Wpi, bpi):
    m = _build_adj(edge_index)
    xt = x.reshape(_G, _N, 128)
    xt = _gat_layer(xt, m, W1, a_s1, a_d1, b1, 4, 64, False)
    xt = _gat_layer(xt, m, W2, a_s2, a_d2, b2, 4, 64, False)
    gemb = _gat_layer(xt, m, W3, a_s3, a_d3, b3, 1, 64, True)

    wout = jnp.concatenate([Wmu, Wlv, Wpi], axis=0).T
    bout = jnp.concatenate([bmu, blv, bpi], axis=0)
    out = _lstm_heads(gemb, Wih_f.T, Whh_f.T, bih_f + bhh_f,
                      Wih_b.T, Whh_b.T, bih_b + bhh_b, wout, bout)
    mu = out[:, :2048].reshape(4, 32, 64)
    logvar = out[:, 2048:4096].reshape(4, 32, 64)
    pi = out[:, 4096:4128]
    return (mu, logvar, pi)
